# baseline (speedup 1.0000x reference)
import functools

import jax
import jax.numpy as jnp
from jax import lax
from jax.experimental import pallas as pl
from jax.experimental.pallas import tpu as pltpu
from jax.experimental.pallas import tpu_sc as plsc

N_PAD = 10240
TILES = 16
WIN = N_PAD // TILES
ACC_ROWS = WIN + 8
K = 128
IDXBLK = 2048
SUBV = IDXBLK // 16
STAGE = 272
BLK = 1024
GRID = N_PAD // BLK
H = 128
G = 64


def _sc_edge_aggregate(src1, dst1, ehalf, nblk, hs=None):
  gather = hs is not None
  width = H if gather else 16
  mesh = plsc.VectorSubcoreMesh(core_axis_name="c", subcore_axis_name="s")

  scratch = [
      pltpu.VMEM((IDXBLK,), jnp.int32),
      pltpu.VMEM((STAGE,), jnp.int32),
      pltpu.VMEM((1, K), jnp.int32),
      pltpu.VMEM((64, width), jnp.float32),
      pltpu.VMEM_SHARED((ACC_ROWS, width), jnp.float32),
      pltpu.SemaphoreType.DMA,
  ]
  if gather:
    scratch += [
        pltpu.VMEM((IDXBLK,), jnp.int32),
        pltpu.VMEM((STAGE,), jnp.int32),
        pltpu.VMEM((1, K), jnp.int32),
        pltpu.VMEM((K, H), jnp.float32),
    ]
  else:
    scratch += [pltpu.VMEM((K, 16), jnp.float32)]

  def body(*refs):
    if gather:
      (hs_hbm, src_hbm, dst_hbm, out_hbm, dbig, stage_d, fire_d, zbuf, acc,
       sem, sbig, stage_s, fire_s, rows) = refs
    else:
      (dst_hbm, out_hbm, dbig, stage_d, fire_d, zbuf, acc, sem,
       ones_v) = refs
    c = lax.axis_index("c")
    s = lax.axis_index("s")
    base = s * WIN
    zero16 = jnp.zeros((16,), jnp.float32)

    def zrow(r, carry):
      for j in range(width // 16):
        zbuf[r, pl.ds(j * 16, 16)] = zero16
      return carry

    lax.fori_loop(0, 64, zrow, 0)
    if not gather:
      lane = lax.iota(jnp.int32, 16)
      e1 = jnp.where(lane == 0, 1.0, 0.0).astype(jnp.float32)

      def orow(r, carry):
        ones_v[r, :] = e1
        return carry

      lax.fori_loop(0, K, orow, 0)

    def zacc(i, carry):
      pltpu.sync_copy(zbuf, acc.at[pl.ds(i * 64, 64)])
      return carry

    lax.fori_loop(0, WIN // 64, zacc, 0)

    def fire_copy():
      for k in range(K // 16):
        fire_d[0, pl.ds(k * 16, 16)] = stage_d[pl.ds(k * 16, 16)]
      for k in range(K // 16):
        stage_d[pl.ds(k * 16, 16)] = stage_d[pl.ds(K + k * 16, 16)]
      if gather:
        for k in range(K // 16):
          fire_s[0, pl.ds(k * 16, 16)] = stage_s[pl.ds(k * 16, 16)]
        for k in range(K // 16):
          stage_s[pl.ds(k * 16, 16)] = stage_s[pl.ds(K + k * 16, 16)]

    def drain():
      pltpu.make_async_copy(hs_hbm.at[fire_s.at[0]], rows, sem).wait()
      pltpu.sync_copy(rows, acc.at[fire_d.at[0]], add=True)

    def blk_body(g, cnt):
      off = c * ehalf + g * IDXBLK
      pltpu.sync_copy(dst_hbm.at[pl.ds(off, IDXBLK)], dbig)
      if gather:
        pltpu.sync_copy(src_hbm.at[pl.ds(off, IDXBLK)], sbig)

      base_v = jnp.broadcast_to(base, (16,)).astype(jnp.int32)
      win_v = jnp.full((16,), WIN, jnp.int32)
      zero_v = jnp.zeros((16,), jnp.int32)

      def sub(j, carry):
        cnt, pend = carry
        dv = dbig[pl.ds(j * 16, 16)]
        dl = dv - base_v
        m = (dl >= zero_v) & (dl < win_v)
        n = plsc.all_reduce_population_count(m)[0]

        @pl.when(n > 0)
        def _():
          plsc.store_compressed(stage_d.at[pl.ds(cnt, 16)], dl, mask=m)
          if gather:
            sv = sbig[pl.ds(j * 16, 16)]
            plsc.store_compressed(stage_s.at[pl.ds(cnt, 16)], sv, mask=m)

        cnt = cnt + n

        @pl.when(cnt >= K)
        def _():
          if gather:
            @pl.when(pend == 1)
            def _():
              drain()

            fire_copy()
            pltpu.async_copy(hs_hbm.at[fire_s.at[0]], rows, sem)
            drain()
          else:
            fire_copy()
            pltpu.sync_copy(ones_v, acc.at[fire_d.at[0]], add=True)

        fired = cnt >= K
        cnt = jnp.where(fired, cnt - K, cnt)
        return cnt, pend

      return lax.fori_loop(0, SUBV, sub, cnt)

    cnt, pend = lax.fori_loop(
        0, nblk, blk_body, (jnp.int32(0), jnp.int32(0)))

    if gather:

      @pl.when(pend == 1)
      def _():
        drain()

    trash = jnp.full((16,), WIN, jnp.int32)
    zeroi = jnp.zeros((16,), jnp.int32)
    for k in range(K // 16):
      stage_d[pl.ds(cnt + k * 16, 16)] = trash
      if gather:
        stage_s[pl.ds(cnt + k * 16, 16)] = zeroi

    @pl.when(cnt > 0)
    def _():
      fire_copy()
      if gather:
        pltpu.async_copy(hs_hbm.at[fire_s.at[0]], rows, sem)
        drain()
      else:
        pltpu.sync_copy(ones_v, acc.at[fire_d.at[0]], add=True)

    pltpu.sync_copy(
        acc.at[pl.ds(0, WIN)],
        out_hbm.at[pl.ds(c * N_PAD + base, WIN)],
    )

  kern = functools.partial(
      pl.kernel,
      mesh=mesh,
      out_type=jax.ShapeDtypeStruct((2 * N_PAD, width), jnp.float32),
      scratch_types=scratch,
      compiler_params=pltpu.CompilerParams(needs_layout_passes=False),
  )(body)
  args = (hs, src1, dst1) if gather else (dst1,)
  return kern(*args).reshape(2, N_PAD, width)


def _tc_layer0(x, W, d0, d1):
  def body(x_ref, w_ref, d0_ref, d1_ref, o_ref):
    dinv = lax.rsqrt(d0_ref[:, 0:1] + d1_ref[:, 0:1] + 1.0)
    h = jnp.dot(x_ref[...], w_ref[...], preferred_element_type=jnp.float32)
    o_ref[...] = h * dinv

  return pl.pallas_call(
      body,
      grid=(GRID,),
      in_specs=[
          pl.BlockSpec((BLK, H), lambda i: (i, 0)),
          pl.BlockSpec((H, H), lambda i: (0, 0)),
          pl.BlockSpec((BLK, 16), lambda i: (i, 0)),
          pl.BlockSpec((BLK, 16), lambda i: (i, 0)),
      ],
      out_specs=pl.BlockSpec((BLK, H), lambda i: (i, 0)),
      out_shape=jax.ShapeDtypeStruct((N_PAD, H), jnp.float32),
  )(x, W, d0, d1)


def _tc_layer(p0, p1, hs, d0, d1, b, W):
  def body(p0_ref, p1_ref, hs_ref, d0_ref, d1_ref, b_ref, w_ref, o_ref):
    dinv = lax.rsqrt(d0_ref[:, 0:1] + d1_ref[:, 0:1] + 1.0)
    t = (p0_ref[...] + p1_ref[...] + hs_ref[...]) * dinv + b_ref[...]
    xn = jnp.maximum(t, 0.0)
    h = jnp.dot(xn, w_ref[...], preferred_element_type=jnp.float32)
    o_ref[...] = h * dinv

  return pl.pallas_call(
      body,
      grid=(GRID,),
      in_specs=[
          pl.BlockSpec((BLK, H), lambda i: (i, 0)),
          pl.BlockSpec((BLK, H), lambda i: (i, 0)),
          pl.BlockSpec((BLK, H), lambda i: (i, 0)),
          pl.BlockSpec((BLK, 16), lambda i: (i, 0)),
          pl.BlockSpec((BLK, 16), lambda i: (i, 0)),
          pl.BlockSpec((1, H), lambda i: (0, 0)),
          pl.BlockSpec((H, H), lambda i: (0, 0)),
      ],
      out_specs=pl.BlockSpec((BLK, H), lambda i: (i, 0)),
      out_shape=jax.ShapeDtypeStruct((N_PAD, H), jnp.float32),
  )(p0, p1, hs, d0, d1, b, W)


def _tc_final(p0, p1, hs, d0, d1, b, batf3, Wcp, bcp):
  def body(p0_ref, p1_ref, hs_ref, d0_ref, d1_ref, b_ref, bat_ref, wc_ref,
           bc_ref, o_ref, ssum, cnt):
    i = pl.program_id(0)

    @pl.when(i == 0)
    def _init():
      ssum[...] = jnp.zeros((G, H), jnp.float32)
      cnt[...] = jnp.zeros((G, H), jnp.float32)

    dinv = lax.rsqrt(d0_ref[:, 0:1] + d1_ref[:, 0:1] + 1.0)
    t = (p0_ref[...] + p1_ref[...] + hs_ref[...]) * dinv + b_ref[...]
    x3 = jnp.maximum(t, 0.0)
    bat = bat_ref[...].reshape(1, BLK)
    gid = lax.broadcasted_iota(jnp.int32, (G, BLK), 0).astype(jnp.float32)
    mask = jnp.where(gid == bat, 1.0, 0.0)
    ssum[...] += jnp.dot(mask, x3, preferred_element_type=jnp.float32)
    cnt[...] += jnp.broadcast_to(
        jnp.sum(mask, axis=1, keepdims=True), (G, H))

    @pl.when(i == GRID - 1)
    def _fin():
      pooled = ssum[...] / jnp.maximum(cnt[...], 1.0)
      o_ref[...] = (
          jnp.dot(pooled, wc_ref[...], preferred_element_type=jnp.float32)
          + bc_ref[...])

  return pl.pallas_call(
      body,
      grid=(GRID,),
      in_specs=[
          pl.BlockSpec((BLK, H), lambda i: (i, 0)),
          pl.BlockSpec((BLK, H), lambda i: (i, 0)),
          pl.BlockSpec((BLK, H), lambda i: (i, 0)),
          pl.BlockSpec((BLK, 16), lambda i: (i, 0)),
          pl.BlockSpec((BLK, 16), lambda i: (i, 0)),
          pl.BlockSpec((1, H), lambda i: (0, 0)),
          pl.BlockSpec((1, 1, BLK), lambda i: (i, 0, 0)),
          pl.BlockSpec((H, H), lambda i: (0, 0)),
          pl.BlockSpec((1, H), lambda i: (0, 0)),
      ],
      out_specs=pl.BlockSpec((G, H), lambda i: (0, 0)),
      out_shape=jax.ShapeDtypeStruct((G, H), jnp.float32),
      scratch_shapes=[
          pltpu.VMEM((G, H), jnp.float32),
          pltpu.VMEM((G, H), jnp.float32),
      ],
  )(p0, p1, hs, d0, d1, b, batf3, Wcp, bcp)


def kernel(x, edge_index, batch, W0, b0, W1, b1, W2, b2, Wc, bc):
  N = x.shape[0]
  E = edge_index.shape[1]
  C = Wc.shape[1]
  src = edge_index[0]
  dst = edge_index[1]

  nblk = -(-E // (2 * IDXBLK))
  epad = 2 * IDXBLK * nblk - E
  ehalf = IDXBLK * nblk
  srcp = jnp.concatenate([src, jnp.zeros((epad,), jnp.int32)])
  dstp = jnp.concatenate([dst, jnp.full((epad,), N_PAD, jnp.int32)])

  xp = jnp.pad(x.astype(jnp.float32), ((0, N_PAD - N), (0, 0)))
  batf3 = jnp.pad(
      batch.astype(jnp.float32), (0, N_PAD - N),
      constant_values=1e9).reshape(GRID, 1, BLK)
  b0r = b0.reshape(1, H)
  b1r = b1.reshape(1, H)
  b2r = b2.reshape(1, H)
  Wcp = jnp.pad(Wc, ((0, 0), (0, H - C)))
  bcp = jnp.pad(bc, (0, H - C)).reshape(1, H)

  degp = _sc_edge_aggregate(srcp, dstp, ehalf, nblk)
  d0, d1 = degp[0], degp[1]

  hs0 = _tc_layer0(xp, W0, d0, d1)
  p = _sc_edge_aggregate(srcp, dstp, ehalf, nblk, hs=hs0)
  hs1 = _tc_layer(p[0], p[1], hs0, d0, d1, b0r, W1)
  p = _sc_edge_aggregate(srcp, dstp, ehalf, nblk, hs=hs1)
  hs2 = _tc_layer(p[0], p[1], hs1, d0, d1, b1r, W2)
  p = _sc_edge_aggregate(srcp, dstp, ehalf, nblk, hs=hs2)
  out = _tc_final(p[0], p[1], hs2, d0, d1, b2r, batf3, Wcp, bcp)
  return out[:, :C]


# speedup vs baseline: 4.2051x; 4.2051x over previous
import functools

import jax
import jax.numpy as jnp
from jax import lax
from jax.experimental import pallas as pl
from jax.experimental.pallas import tpu as pltpu
from jax.experimental.pallas import tpu_sc as plsc

N_PAD = 10240
TILES = 16
NW = 2 * TILES
WIN = N_PAD // TILES
ACC_ROWS = WIN + 8
K = 128
IDXBLK = 2048
SUBV = IDXBLK // 16
STAGE = 272
BLK = 1024
GRID = N_PAD // BLK
H = 128
G = 64

_SC_PARAMS = pltpu.CompilerParams(needs_layout_passes=False)
_MESH = dict(core_axis_name="c", subcore_axis_name="s")


def _sc_route(src1, dst1, ehalf, nblk):
  mesh = plsc.VectorSubcoreMesh(**_MESH)

  @functools.partial(
      pl.kernel,
      mesh=mesh,
      out_type=(
          jax.ShapeDtypeStruct((2 * N_PAD, 16), jnp.float32),
          jax.ShapeDtypeStruct((NW * ehalf,), jnp.int32),
          jax.ShapeDtypeStruct((NW * ehalf,), jnp.int32),
          jax.ShapeDtypeStruct((NW * 16,), jnp.int32),
      ),
      scratch_types=[
          pltpu.VMEM((IDXBLK,), jnp.int32),
          pltpu.VMEM((IDXBLK,), jnp.int32),
          pltpu.VMEM((STAGE,), jnp.int32),
          pltpu.VMEM((STAGE,), jnp.int32),
          pltpu.VMEM((1, K), jnp.int32),
          pltpu.VMEM((1, K), jnp.int32),
          pltpu.VMEM((K, 16), jnp.float32),
          pltpu.VMEM((64, 16), jnp.float32),
          pltpu.VMEM((16,), jnp.int32),
          pltpu.VMEM_SHARED((ACC_ROWS, 16), jnp.float32),
      ],
      compiler_params=_SC_PARAMS,
  )
  def route_kernel(src_hbm, dst_hbm, deg_hbm, slist_hbm, dlist_hbm,
                   counts_hbm, dbig, sbig, stage_d, stage_s, fire_d, fire_s,
                   ones_v, zbuf, cbuf, acc):
    c = lax.axis_index("c")
    s = lax.axis_index("s")
    wid = c * TILES + s
    base = s * WIN
    lbase = wid * ehalf
    zero16 = jnp.zeros((16,), jnp.float32)
    lane = lax.iota(jnp.int32, 16)
    e1 = jnp.where(lane == 0, 1.0, 0.0).astype(jnp.float32)

    def initrow(r, carry):
      zbuf[r, :] = zero16
      return carry

    lax.fori_loop(0, 64, initrow, 0)

    def orow(r, carry):
      ones_v[r, :] = e1
      return carry

    lax.fori_loop(0, K, orow, 0)

    def zacc(i, carry):
      pltpu.sync_copy(zbuf, acc.at[pl.ds(i * 64, 64)])
      return carry

    lax.fori_loop(0, WIN // 64, zacc, 0)

    def fire_copy():
      for k in range(K // 16):
        fire_d[0, pl.ds(k * 16, 16)] = stage_d[pl.ds(k * 16, 16)]
      for k in range(K // 16):
        stage_d[pl.ds(k * 16, 16)] = stage_d[pl.ds(K + k * 16, 16)]
      for k in range(K // 16):
        fire_s[0, pl.ds(k * 16, 16)] = stage_s[pl.ds(k * 16, 16)]
      for k in range(K // 16):
        stage_s[pl.ds(k * 16, 16)] = stage_s[pl.ds(K + k * 16, 16)]

    def fire_out(nf):
      pltpu.sync_copy(ones_v, acc.at[fire_d.at[0]], add=True)
      pltpu.sync_copy(fire_s.at[0], slist_hbm.at[pl.ds(lbase + nf * K, K)])
      pltpu.sync_copy(fire_d.at[0], dlist_hbm.at[pl.ds(lbase + nf * K, K)])

    def blk_body(g, carry):
      off = c * ehalf + g * IDXBLK
      pltpu.sync_copy(dst_hbm.at[pl.ds(off, IDXBLK)], dbig)
      pltpu.sync_copy(src_hbm.at[pl.ds(off, IDXBLK)], sbig)
      base_v = jnp.broadcast_to(base, (16,)).astype(jnp.int32)
      win_v = jnp.full((16,), WIN, jnp.int32)
      zero_v = jnp.zeros((16,), jnp.int32)

      def sub(j, carry):
        cnt, nf = carry
        dv = dbig[pl.ds(j * 16, 16)]
        dl = dv - base_v
        m = (dl >= zero_v) & (dl < win_v)
        n = plsc.all_reduce_population_count(m)[0]
        plsc.store_compressed(stage_d.at[pl.ds(cnt, 16)], dl, mask=m)
        sv = sbig[pl.ds(j * 16, 16)]
        plsc.store_compressed(stage_s.at[pl.ds(cnt, 16)], sv, mask=m)
        cnt = cnt + n

        @pl.when(cnt >= K)
        def _():
          fire_copy()
          fire_out(nf)

        fired = cnt >= K
        cnt = jnp.where(fired, cnt - K, cnt)
        nf = jnp.where(fired, nf + 1, nf)
        return cnt, nf

      return lax.fori_loop(0, SUBV, sub, carry)

    cnt, nf = lax.fori_loop(
        0, nblk, blk_body, (jnp.int32(0), jnp.int32(0)))

    trash = jnp.full((16,), WIN, jnp.int32)
    zeroi = jnp.zeros((16,), jnp.int32)
    for k in range(K // 16):
      stage_d[pl.ds(cnt + k * 16, 16)] = trash
      stage_s[pl.ds(cnt + k * 16, 16)] = zeroi

    @pl.when(cnt > 0)
    def _():
      fire_copy()
      fire_out(nf)

    nf = jnp.where(cnt > 0, nf + 1, nf)

    cbuf[...] = jnp.broadcast_to(nf, (16,)).astype(jnp.int32)
    pltpu.sync_copy(cbuf, counts_hbm.at[pl.ds(wid * 16, 16)])
    pltpu.sync_copy(
        acc.at[pl.ds(0, WIN)],
        deg_hbm.at[pl.ds(c * N_PAD + base, WIN)],
    )

  deg, slist, dlist, counts = route_kernel(src1, dst1)
  return deg.reshape(2, N_PAD, 16), slist, dlist, counts


def _sc_apply(hs, slist, dlist, degflat, ehalf):
  mesh = plsc.VectorSubcoreMesh(**_MESH)

  @functools.partial(
      pl.kernel,
      mesh=mesh,
      out_type=jax.ShapeDtypeStruct((2 * N_PAD, H), jnp.float32),
      scratch_types=[
          pltpu.VMEM((1, K), jnp.int32),
          pltpu.VMEM((1, K), jnp.int32),
          pltpu.VMEM((K, H), jnp.float32),
          pltpu.VMEM((64, H), jnp.float32),
          pltpu.VMEM((WIN, 16), jnp.float32),
          pltpu.VMEM_SHARED((ACC_ROWS, H), jnp.float32),
          pltpu.SemaphoreType.DMA,
      ],
      compiler_params=_SC_PARAMS,
  )
  def apply_kernel(hs_hbm, slist_hbm, dlist_hbm, deg_hbm, out_hbm,
                   fire_s, fire_d, rows, zbuf, degbuf, acc, sem):
    c = lax.axis_index("c")
    s = lax.axis_index("s")
    wid = c * TILES + s
    base = s * WIN
    lbase = wid * ehalf
    zero16 = jnp.zeros((16,), jnp.float32)

    def zrow(r, carry):
      for j in range(H // 16):
        zbuf[r, pl.ds(j * 16, 16)] = zero16
      return carry

    lax.fori_loop(0, 64, zrow, 0)

    def zacc(i, carry):
      pltpu.sync_copy(zbuf, acc.at[pl.ds(i * 64, 64)])
      return carry

    lax.fori_loop(0, WIN // 64, zacc, 0)

    pltpu.sync_copy(deg_hbm.at[pl.ds(c * N_PAD + base, WIN)], degbuf)

    def dsum(r, tv):
      return tv + degbuf[r, :]

    tv = lax.fori_loop(0, WIN, dsum, jnp.zeros((16,), jnp.float32))
    matched = jnp.sum(tv).astype(jnp.int32)
    nf = (matched + K - 1) // K

    def fire(f, carry):
      @pl.when(f < nf)
      def _():
        pltpu.sync_copy(slist_hbm.at[pl.ds(lbase + f * K, K)], fire_s.at[0])
        pltpu.sync_copy(dlist_hbm.at[pl.ds(lbase + f * K, K)], fire_d.at[0])
        pltpu.async_copy(hs_hbm.at[fire_s.at[0]], rows, sem).wait()
        pltpu.sync_copy(rows, acc.at[fire_d.at[0]], add=True)
      return carry

    lax.fori_loop(0, ehalf // K, fire, 0)

    pltpu.sync_copy(
        acc.at[pl.ds(0, WIN)],
        out_hbm.at[pl.ds(c * N_PAD + base, WIN)],
    )

  return apply_kernel(hs, slist, dlist, degflat).reshape(2, N_PAD, H)


def _tc_layer0(x, W, d0, d1):
  def body(x_ref, w_ref, d0_ref, d1_ref, o_ref):
    dinv = lax.rsqrt(d0_ref[:, 0:1] + d1_ref[:, 0:1] + 1.0)
    h = jnp.dot(x_ref[...], w_ref[...], preferred_element_type=jnp.float32)
    o_ref[...] = h * dinv

  return pl.pallas_call(
      body,
      grid=(GRID,),
      in_specs=[
          pl.BlockSpec((BLK, H), lambda i: (i, 0)),
          pl.BlockSpec((H, H), lambda i: (0, 0)),
          pl.BlockSpec((BLK, 16), lambda i: (i, 0)),
          pl.BlockSpec((BLK, 16), lambda i: (i, 0)),
      ],
      out_specs=pl.BlockSpec((BLK, H), lambda i: (i, 0)),
      out_shape=jax.ShapeDtypeStruct((N_PAD, H), jnp.float32),
  )(x, W, d0, d1)


def _tc_layer(p0, p1, hs, d0, d1, b, W):
  def body(p0_ref, p1_ref, hs_ref, d0_ref, d1_ref, b_ref, w_ref, o_ref):
    dinv = lax.rsqrt(d0_ref[:, 0:1] + d1_ref[:, 0:1] + 1.0)
    t = (p0_ref[...] + p1_ref[...] + hs_ref[...]) * dinv + b_ref[...]
    xn = jnp.maximum(t, 0.0)
    h = jnp.dot(xn, w_ref[...], preferred_element_type=jnp.float32)
    o_ref[...] = h * dinv

  return pl.pallas_call(
      body,
      grid=(GRID,),
      in_specs=[
          pl.BlockSpec((BLK, H), lambda i: (i, 0)),
          pl.BlockSpec((BLK, H), lambda i: (i, 0)),
          pl.BlockSpec((BLK, H), lambda i: (i, 0)),
          pl.BlockSpec((BLK, 16), lambda i: (i, 0)),
          pl.BlockSpec((BLK, 16), lambda i: (i, 0)),
          pl.BlockSpec((1, H), lambda i: (0, 0)),
          pl.BlockSpec((H, H), lambda i: (0, 0)),
      ],
      out_specs=pl.BlockSpec((BLK, H), lambda i: (i, 0)),
      out_shape=jax.ShapeDtypeStruct((N_PAD, H), jnp.float32),
  )(p0, p1, hs, d0, d1, b, W)


def _tc_final(p0, p1, hs, d0, d1, b, batf3, Wcp, bcp):
  def body(p0_ref, p1_ref, hs_ref, d0_ref, d1_ref, b_ref, bat_ref, wc_ref,
           bc_ref, o_ref, ssum, cnt):
    i = pl.program_id(0)

    @pl.when(i == 0)
    def _init():
      ssum[...] = jnp.zeros((G, H), jnp.float32)
      cnt[...] = jnp.zeros((G, H), jnp.float32)

    dinv = lax.rsqrt(d0_ref[:, 0:1] + d1_ref[:, 0:1] + 1.0)
    t = (p0_ref[...] + p1_ref[...] + hs_ref[...]) * dinv + b_ref[...]
    x3 = jnp.maximum(t, 0.0)
    bat = bat_ref[...].reshape(1, BLK)
    gid = lax.broadcasted_iota(jnp.int32, (G, BLK), 0).astype(jnp.float32)
    mask = jnp.where(gid == bat, 1.0, 0.0)
    ssum[...] += jnp.dot(mask, x3, preferred_element_type=jnp.float32)
    cnt[...] += jnp.broadcast_to(
        jnp.sum(mask, axis=1, keepdims=True), (G, H))

    @pl.when(i == GRID - 1)
    def _fin():
      pooled = ssum[...] / jnp.maximum(cnt[...], 1.0)
      o_ref[...] = (
          jnp.dot(pooled, wc_ref[...], preferred_element_type=jnp.float32)
          + bc_ref[...])

  return pl.pallas_call(
      body,
      grid=(GRID,),
      in_specs=[
          pl.BlockSpec((BLK, H), lambda i: (i, 0)),
          pl.BlockSpec((BLK, H), lambda i: (i, 0)),
          pl.BlockSpec((BLK, H), lambda i: (i, 0)),
          pl.BlockSpec((BLK, 16), lambda i: (i, 0)),
          pl.BlockSpec((BLK, 16), lambda i: (i, 0)),
          pl.BlockSpec((1, H), lambda i: (0, 0)),
          pl.BlockSpec((1, 1, BLK), lambda i: (i, 0, 0)),
          pl.BlockSpec((H, H), lambda i: (0, 0)),
          pl.BlockSpec((1, H), lambda i: (0, 0)),
      ],
      out_specs=pl.BlockSpec((G, H), lambda i: (0, 0)),
      out_shape=jax.ShapeDtypeStruct((G, H), jnp.float32),
      scratch_shapes=[
          pltpu.VMEM((G, H), jnp.float32),
          pltpu.VMEM((G, H), jnp.float32),
      ],
  )(p0, p1, hs, d0, d1, b, batf3, Wcp, bcp)


def kernel(x, edge_index, batch, W0, b0, W1, b1, W2, b2, Wc, bc):
  N = x.shape[0]
  E = edge_index.shape[1]
  C = Wc.shape[1]
  src = edge_index[0]
  dst = edge_index[1]

  nblk = -(-E // (2 * IDXBLK))
  epad = 2 * IDXBLK * nblk - E
  ehalf = IDXBLK * nblk
  srcp = jnp.concatenate([src, jnp.zeros((epad,), jnp.int32)])
  dstp = jnp.concatenate([dst, jnp.full((epad,), N_PAD, jnp.int32)])

  xp = jnp.pad(x.astype(jnp.float32), ((0, N_PAD - N), (0, 0)))
  batf3 = jnp.pad(
      batch.astype(jnp.float32), (0, N_PAD - N),
      constant_values=1e9).reshape(GRID, 1, BLK)
  b0r = b0.reshape(1, H)
  b1r = b1.reshape(1, H)
  b2r = b2.reshape(1, H)
  Wcp = jnp.pad(Wc, ((0, 0), (0, H - C)))
  bcp = jnp.pad(bc, (0, H - C)).reshape(1, H)

  degp, slist, dlist, counts = _sc_route(srcp, dstp, ehalf, nblk)
  del counts
  degflat = degp.reshape(2 * N_PAD, 16)
  d0, d1 = degp[0], degp[1]

  hs0 = _tc_layer0(xp, W0, d0, d1)
  p = _sc_apply(hs0, slist, dlist, degflat, ehalf)
  hs1 = _tc_layer(p[0], p[1], hs0, d0, d1, b0r, W1)
  p = _sc_apply(hs1, slist, dlist, degflat, ehalf)
  hs2 = _tc_layer(p[0], p[1], hs1, d0, d1, b1r, W2)
  p = _sc_apply(hs2, slist, dlist, degflat, ehalf)
  out = _tc_final(p[0], p[1], hs2, d0, d1, b2r, batf3, Wcp, bcp)
  return out[:, :C]


# speedup vs baseline: 4.3794x; 1.0414x over previous
import functools

import jax
import jax.numpy as jnp
from jax import lax
from jax.experimental import pallas as pl
from jax.experimental.pallas import tpu as pltpu
from jax.experimental.pallas import tpu_sc as plsc

N_PAD = 10240
TILES = 16
NW = 2 * TILES
WIN = N_PAD // TILES
ACC_ROWS = WIN + 8
K = 128
IDXBLK = 2048
SUBV = IDXBLK // 16
STAGE = 272
BLK = 1024
GRID = N_PAD // BLK
H = 128
G = 64

_SC_PARAMS = pltpu.CompilerParams(needs_layout_passes=False)
_MESH = dict(core_axis_name="c", subcore_axis_name="s")


def _sc_route(src1, dst1, ehalf, nblk):
  mesh = plsc.VectorSubcoreMesh(**_MESH)

  @functools.partial(
      pl.kernel,
      mesh=mesh,
      out_type=(
          jax.ShapeDtypeStruct((2 * N_PAD, 16), jnp.float32),
          jax.ShapeDtypeStruct((NW * ehalf,), jnp.int32),
          jax.ShapeDtypeStruct((NW * ehalf,), jnp.int32),
          jax.ShapeDtypeStruct((NW * 16,), jnp.int32),
      ),
      scratch_types=[
          pltpu.VMEM((IDXBLK,), jnp.int32),
          pltpu.VMEM((IDXBLK,), jnp.int32),
          pltpu.VMEM((STAGE,), jnp.int32),
          pltpu.VMEM((STAGE,), jnp.int32),
          pltpu.VMEM((1, K), jnp.int32),
          pltpu.VMEM((1, K), jnp.int32),
          pltpu.VMEM((K, 16), jnp.float32),
          pltpu.VMEM((64, 16), jnp.float32),
          pltpu.VMEM((16,), jnp.int32),
          pltpu.VMEM_SHARED((ACC_ROWS, 16), jnp.float32),
      ],
      compiler_params=_SC_PARAMS,
  )
  def route_kernel(src_hbm, dst_hbm, deg_hbm, slist_hbm, dlist_hbm,
                   counts_hbm, dbig, sbig, stage_d, stage_s, fire_d, fire_s,
                   ones_v, zbuf, cbuf, acc):
    c = lax.axis_index("c")
    s = lax.axis_index("s")
    wid = c * TILES + s
    base = s * WIN
    lbase = wid * ehalf
    zero16 = jnp.zeros((16,), jnp.float32)
    lane = lax.iota(jnp.int32, 16)
    e1 = jnp.where(lane == 0, 1.0, 0.0).astype(jnp.float32)

    def initrow(r, carry):
      zbuf[r, :] = zero16
      return carry

    lax.fori_loop(0, 64, initrow, 0)

    def orow(r, carry):
      ones_v[r, :] = e1
      return carry

    lax.fori_loop(0, K, orow, 0)

    def zacc(i, carry):
      pltpu.sync_copy(zbuf, acc.at[pl.ds(i * 64, 64)])
      return carry

    lax.fori_loop(0, WIN // 64, zacc, 0)

    def fire_copy():
      for k in range(K // 16):
        fire_d[0, pl.ds(k * 16, 16)] = stage_d[pl.ds(k * 16, 16)]
      for k in range(K // 16):
        stage_d[pl.ds(k * 16, 16)] = stage_d[pl.ds(K + k * 16, 16)]
      for k in range(K // 16):
        fire_s[0, pl.ds(k * 16, 16)] = stage_s[pl.ds(k * 16, 16)]
      for k in range(K // 16):
        stage_s[pl.ds(k * 16, 16)] = stage_s[pl.ds(K + k * 16, 16)]

    def fire_out(nf):
      pltpu.sync_copy(ones_v, acc.at[fire_d.at[0]], add=True)
      pltpu.sync_copy(fire_s.at[0], slist_hbm.at[pl.ds(lbase + nf * K, K)])
      pltpu.sync_copy(fire_d.at[0], dlist_hbm.at[pl.ds(lbase + nf * K, K)])

    def blk_body(g, carry):
      off = c * ehalf + g * IDXBLK
      pltpu.sync_copy(dst_hbm.at[pl.ds(off, IDXBLK)], dbig)
      pltpu.sync_copy(src_hbm.at[pl.ds(off, IDXBLK)], sbig)
      base_v = jnp.broadcast_to(base, (16,)).astype(jnp.int32)
      win_v = jnp.full((16,), WIN, jnp.int32)
      zero_v = jnp.zeros((16,), jnp.int32)

      def sub(j, carry):
        cnt, nf = carry
        dv = dbig[pl.ds(j * 16, 16)]
        dl = dv - base_v
        m = (dl >= zero_v) & (dl < win_v)
        n = plsc.all_reduce_population_count(m)[0]
        plsc.store_compressed(stage_d.at[pl.ds(cnt, 16)], dl, mask=m)
        sv = sbig[pl.ds(j * 16, 16)]
        plsc.store_compressed(stage_s.at[pl.ds(cnt, 16)], sv, mask=m)
        cnt = cnt + n

        @pl.when(cnt >= K)
        def _():
          fire_copy()
          fire_out(nf)

        fired = cnt >= K
        cnt = jnp.where(fired, cnt - K, cnt)
        nf = jnp.where(fired, nf + 1, nf)
        return cnt, nf

      return lax.fori_loop(0, SUBV, sub, carry, unroll=4)

    cnt, nf = lax.fori_loop(
        0, nblk, blk_body, (jnp.int32(0), jnp.int32(0)))

    trash = jnp.full((16,), WIN, jnp.int32)
    zeroi = jnp.zeros((16,), jnp.int32)
    for k in range(K // 16):
      stage_d[pl.ds(cnt + k * 16, 16)] = trash
      stage_s[pl.ds(cnt + k * 16, 16)] = zeroi

    @pl.when(cnt > 0)
    def _():
      fire_copy()
      fire_out(nf)

    nf = jnp.where(cnt > 0, nf + 1, nf)

    cbuf[...] = jnp.broadcast_to(nf, (16,)).astype(jnp.int32)
    pltpu.sync_copy(cbuf, counts_hbm.at[pl.ds(wid * 16, 16)])
    pltpu.sync_copy(
        acc.at[pl.ds(0, WIN)],
        deg_hbm.at[pl.ds(c * N_PAD + base, WIN)],
    )

  deg, slist, dlist, counts = route_kernel(src1, dst1)
  return deg.reshape(2, N_PAD, 16), slist, dlist, counts


def _sc_apply(hs, slist, dlist, degflat, ehalf):
  mesh = plsc.VectorSubcoreMesh(**_MESH)

  @functools.partial(
      pl.kernel,
      mesh=mesh,
      out_type=jax.ShapeDtypeStruct((2 * N_PAD, H), jnp.float32),
      scratch_types=[
          pltpu.VMEM((1, K), jnp.int32),
          pltpu.VMEM((1, K), jnp.int32),
          pltpu.VMEM((K, H), jnp.float32),
          pltpu.VMEM((64, H), jnp.float32),
          pltpu.VMEM((WIN, 16), jnp.float32),
          pltpu.VMEM_SHARED((ACC_ROWS, H), jnp.float32),
          pltpu.SemaphoreType.DMA,
      ],
      compiler_params=_SC_PARAMS,
  )
  def apply_kernel(hs_hbm, slist_hbm, dlist_hbm, deg_hbm, out_hbm,
                   fire_s, fire_d, rows, zbuf, degbuf, acc, sem):
    c = lax.axis_index("c")
    s = lax.axis_index("s")
    wid = c * TILES + s
    base = s * WIN
    lbase = wid * ehalf
    zero16 = jnp.zeros((16,), jnp.float32)

    def zrow(r, carry):
      for j in range(H // 16):
        zbuf[r, pl.ds(j * 16, 16)] = zero16
      return carry

    lax.fori_loop(0, 64, zrow, 0)

    def zacc(i, carry):
      pltpu.sync_copy(zbuf, acc.at[pl.ds(i * 64, 64)])
      return carry

    lax.fori_loop(0, WIN // 64, zacc, 0)

    pltpu.sync_copy(deg_hbm.at[pl.ds(c * N_PAD + base, WIN)], degbuf)

    def dsum(r, tv):
      return tv + degbuf[r, :]

    tv = lax.fori_loop(0, WIN, dsum, jnp.zeros((16,), jnp.float32))
    matched = jnp.sum(tv).astype(jnp.int32)
    nf = (matched + K - 1) // K

    def fire(f, carry):
      @pl.when(f < nf)
      def _():
        pltpu.sync_copy(slist_hbm.at[pl.ds(lbase + f * K, K)], fire_s.at[0])
        pltpu.sync_copy(dlist_hbm.at[pl.ds(lbase + f * K, K)], fire_d.at[0])
        pltpu.async_copy(hs_hbm.at[fire_s.at[0]], rows, sem).wait()
        pltpu.sync_copy(rows, acc.at[fire_d.at[0]], add=True)
      return carry

    lax.fori_loop(0, ehalf // K, fire, 0)

    pltpu.sync_copy(
        acc.at[pl.ds(0, WIN)],
        out_hbm.at[pl.ds(c * N_PAD + base, WIN)],
    )

  return apply_kernel(hs, slist, dlist, degflat).reshape(2, N_PAD, H)


def _tc_layer0(x, W, d0, d1):
  def body(x_ref, w_ref, d0_ref, d1_ref, o_ref):
    dinv = lax.rsqrt(d0_ref[:, 0:1] + d1_ref[:, 0:1] + 1.0)
    h = jnp.dot(x_ref[...], w_ref[...], preferred_element_type=jnp.float32)
    o_ref[...] = h * dinv

  return pl.pallas_call(
      body,
      grid=(GRID,),
      in_specs=[
          pl.BlockSpec((BLK, H), lambda i: (i, 0)),
          pl.BlockSpec((H, H), lambda i: (0, 0)),
          pl.BlockSpec((BLK, 16), lambda i: (i, 0)),
          pl.BlockSpec((BLK, 16), lambda i: (i, 0)),
      ],
      out_specs=pl.BlockSpec((BLK, H), lambda i: (i, 0)),
      out_shape=jax.ShapeDtypeStruct((N_PAD, H), jnp.float32),
  )(x, W, d0, d1)


def _tc_layer(p0, p1, hs, d0, d1, b, W):
  def body(p0_ref, p1_ref, hs_ref, d0_ref, d1_ref, b_ref, w_ref, o_ref):
    dinv = lax.rsqrt(d0_ref[:, 0:1] + d1_ref[:, 0:1] + 1.0)
    t = (p0_ref[...] + p1_ref[...] + hs_ref[...]) * dinv + b_ref[...]
    xn = jnp.maximum(t, 0.0)
    h = jnp.dot(xn, w_ref[...], preferred_element_type=jnp.float32)
    o_ref[...] = h * dinv

  return pl.pallas_call(
      body,
      grid=(GRID,),
      in_specs=[
          pl.BlockSpec((BLK, H), lambda i: (i, 0)),
          pl.BlockSpec((BLK, H), lambda i: (i, 0)),
          pl.BlockSpec((BLK, H), lambda i: (i, 0)),
          pl.BlockSpec((BLK, 16), lambda i: (i, 0)),
          pl.BlockSpec((BLK, 16), lambda i: (i, 0)),
          pl.BlockSpec((1, H), lambda i: (0, 0)),
          pl.BlockSpec((H, H), lambda i: (0, 0)),
      ],
      out_specs=pl.BlockSpec((BLK, H), lambda i: (i, 0)),
      out_shape=jax.ShapeDtypeStruct((N_PAD, H), jnp.float32),
  )(p0, p1, hs, d0, d1, b, W)


def _tc_final(p0, p1, hs, d0, d1, b, batf3, Wcp, bcp):
  def body(p0_ref, p1_ref, hs_ref, d0_ref, d1_ref, b_ref, bat_ref, wc_ref,
           bc_ref, o_ref, ssum, cnt):
    i = pl.program_id(0)

    @pl.when(i == 0)
    def _init():
      ssum[...] = jnp.zeros((G, H), jnp.float32)
      cnt[...] = jnp.zeros((G, H), jnp.float32)

    dinv = lax.rsqrt(d0_ref[:, 0:1] + d1_ref[:, 0:1] + 1.0)
    t = (p0_ref[...] + p1_ref[...] + hs_ref[...]) * dinv + b_ref[...]
    x3 = jnp.maximum(t, 0.0)
    bat = bat_ref[...].reshape(1, BLK)
    gid = lax.broadcasted_iota(jnp.int32, (G, BLK), 0).astype(jnp.float32)
    mask = jnp.where(gid == bat, 1.0, 0.0)
    ssum[...] += jnp.dot(mask, x3, preferred_element_type=jnp.float32)
    cnt[...] += jnp.broadcast_to(
        jnp.sum(mask, axis=1, keepdims=True), (G, H))

    @pl.when(i == GRID - 1)
    def _fin():
      pooled = ssum[...] / jnp.maximum(cnt[...], 1.0)
      o_ref[...] = (
          jnp.dot(pooled, wc_ref[...], preferred_element_type=jnp.float32)
          + bc_ref[...])

  return pl.pallas_call(
      body,
      grid=(GRID,),
      in_specs=[
          pl.BlockSpec((BLK, H), lambda i: (i, 0)),
          pl.BlockSpec((BLK, H), lambda i: (i, 0)),
          pl.BlockSpec((BLK, H), lambda i: (i, 0)),
          pl.BlockSpec((BLK, 16), lambda i: (i, 0)),
          pl.BlockSpec((BLK, 16), lambda i: (i, 0)),
          pl.BlockSpec((1, H), lambda i: (0, 0)),
          pl.BlockSpec((1, 1, BLK), lambda i: (i, 0, 0)),
          pl.BlockSpec((H, H), lambda i: (0, 0)),
          pl.BlockSpec((1, H), lambda i: (0, 0)),
      ],
      out_specs=pl.BlockSpec((G, H), lambda i: (0, 0)),
      out_shape=jax.ShapeDtypeStruct((G, H), jnp.float32),
      scratch_shapes=[
          pltpu.VMEM((G, H), jnp.float32),
          pltpu.VMEM((G, H), jnp.float32),
      ],
  )(p0, p1, hs, d0, d1, b, batf3, Wcp, bcp)


def kernel(x, edge_index, batch, W0, b0, W1, b1, W2, b2, Wc, bc):
  N = x.shape[0]
  E = edge_index.shape[1]
  C = Wc.shape[1]
  src = edge_index[0]
  dst = edge_index[1]

  nblk = -(-E // (2 * IDXBLK))
  epad = 2 * IDXBLK * nblk - E
  ehalf = IDXBLK * nblk
  srcp = jnp.concatenate([src, jnp.zeros((epad,), jnp.int32)])
  dstp = jnp.concatenate([dst, jnp.full((epad,), N_PAD, jnp.int32)])

  xp = jnp.pad(x.astype(jnp.float32), ((0, N_PAD - N), (0, 0)))
  batf3 = jnp.pad(
      batch.astype(jnp.float32), (0, N_PAD - N),
      constant_values=1e9).reshape(GRID, 1, BLK)
  b0r = b0.reshape(1, H)
  b1r = b1.reshape(1, H)
  b2r = b2.reshape(1, H)
  Wcp = jnp.pad(Wc, ((0, 0), (0, H - C)))
  bcp = jnp.pad(bc, (0, H - C)).reshape(1, H)

  degp, slist, dlist, counts = _sc_route(srcp, dstp, ehalf, nblk)
  del counts
  degflat = degp.reshape(2 * N_PAD, 16)
  d0, d1 = degp[0], degp[1]

  hs0 = _tc_layer0(xp, W0, d0, d1)
  p = _sc_apply(hs0, slist, dlist, degflat, ehalf)
  hs1 = _tc_layer(p[0], p[1], hs0, d0, d1, b0r, W1)
  p = _sc_apply(hs1, slist, dlist, degflat, ehalf)
  hs2 = _tc_layer(p[0], p[1], hs1, d0, d1, b1r, W2)
  p = _sc_apply(hs2, slist, dlist, degflat, ehalf)
  out = _tc_final(p[0], p[1], hs2, d0, d1, b2r, batf3, Wcp, bcp)
  return out[:, :C]
